# baseline (device time: 213073 ns/iter reference)
import jax
import jax.numpy as jnp
from jax import lax
from jax.experimental import pallas as pl
from jax.experimental.pallas import tpu as pltpu

N_DEV = 16
B_LOC = 2
SQ = 128
SKV = 128
D_MODEL = 512
H_LOC = 4
DH = 64
D_CHUNK = H_LOC * DH


def kernel(x, Wq, K_ext, V_ext, Wo):
    pos = lax.axis_index("i")
    K_loc = lax.dynamic_slice_in_dim(K_ext, pos * B_LOC, B_LOC, axis=0)
    V_loc = lax.dynamic_slice_in_dim(V_ext, pos * B_LOC, B_LOC, axis=0)
    K_t = jnp.transpose(K_loc, (2, 0, 1, 3))
    V_t = jnp.transpose(V_loc, (2, 0, 1, 3))

    def body(x_ref, wq_ref, k_ref, v_ref, wo_ref, out_ref,
             comm_wq, comm_wo, send_wq, recv_wq, send_wo, recv_wo):
        my = lax.axis_index("i")
        left = lax.rem(my + N_DEV - 1, N_DEV)
        right = lax.rem(my + 1, N_DEV)

        barrier = pltpu.get_barrier_semaphore()
        for nbr in (left, right):
            pl.semaphore_signal(barrier, inc=1, device_id=(nbr,),
                                device_id_type=pl.DeviceIdType.MESH)
        pl.semaphore_wait(barrier, 2)

        qblk = lax.broadcasted_iota(jnp.int32, (SQ, SKV), 0) // 64
        kblk = lax.broadcasted_iota(jnp.int32, (SQ, SKV), 1) // 64
        mask = kblk <= qblk

        x2 = x_ref[...].reshape(B_LOC * SQ, D_MODEL)

        def compute_chunk(o, wq_c, wo_c):
            q = jnp.dot(x2, wq_c, preferred_element_type=jnp.float32)
            k_c = k_ref[pl.ds(o * H_LOC, H_LOC)]
            v_c = v_ref[pl.ds(o * H_LOC, H_LOC)]
            for b in range(B_LOC):
                qb = q[b * SQ:(b + 1) * SQ]
                parts = []
                for h in range(H_LOC):
                    qh = qb[:, h * DH:(h + 1) * DH]
                    kh = k_c[h, b]
                    vh = v_c[h, b]
                    s = lax.dot_general(
                        qh, kh, (((1,), (1,)), ((), ())),
                        preferred_element_type=jnp.float32) * 0.125
                    s = jnp.where(mask, s, -1e9)
                    m = jnp.max(s, axis=1, keepdims=True)
                    w = jnp.exp(s - m)
                    w = w / jnp.sum(w, axis=1, keepdims=True)
                    parts.append(lax.dot_general(
                        w, vh, (((1,), (0,)), ((), ())),
                        preferred_element_type=jnp.float32))
                ctx = jnp.concatenate(parts, axis=1)
                out_ref[b] += jnp.dot(ctx, wo_c,
                                      preferred_element_type=jnp.float32)

        out_ref[...] = jnp.zeros_like(out_ref)

        s1_wq = pltpu.make_async_remote_copy(
            src_ref=wq_ref, dst_ref=comm_wq.at[my],
            send_sem=send_wq.at[0], recv_sem=recv_wq.at[my],
            device_id=(right,), device_id_type=pl.DeviceIdType.MESH)
        s1_wo = pltpu.make_async_remote_copy(
            src_ref=wo_ref, dst_ref=comm_wo.at[my],
            send_sem=send_wo.at[0], recv_sem=recv_wo.at[my],
            device_id=(right,), device_id_type=pl.DeviceIdType.MESH)
        s1_wq.start()
        s1_wo.start()

        compute_chunk(my, wq_ref[...], wo_ref[...])

        for hop in range(1, N_DEV):
            o = lax.rem(my - hop + N_DEV, N_DEV)
            rx_wq = pltpu.make_async_remote_copy(
                src_ref=comm_wq.at[o], dst_ref=comm_wq.at[o],
                send_sem=send_wq.at[hop], recv_sem=recv_wq.at[o],
                device_id=(right,), device_id_type=pl.DeviceIdType.MESH)
            rx_wo = pltpu.make_async_remote_copy(
                src_ref=comm_wo.at[o], dst_ref=comm_wo.at[o],
                send_sem=send_wo.at[hop], recv_sem=recv_wo.at[o],
                device_id=(right,), device_id_type=pl.DeviceIdType.MESH)
            rx_wq.wait_recv()
            rx_wo.wait_recv()
            if hop < N_DEV - 1:
                rx_wq.start()
                rx_wo.start()
            compute_chunk(o, comm_wq[o], comm_wo[o])

        for hop in range(N_DEV - 1):
            tx_wq = pltpu.make_async_remote_copy(
                src_ref=comm_wq.at[0], dst_ref=comm_wq.at[0],
                send_sem=send_wq.at[hop], recv_sem=recv_wq.at[0],
                device_id=(right,), device_id_type=pl.DeviceIdType.MESH)
            tx_wo = pltpu.make_async_remote_copy(
                src_ref=comm_wo.at[0], dst_ref=comm_wo.at[0],
                send_sem=send_wo.at[hop], recv_sem=recv_wo.at[0],
                device_id=(right,), device_id_type=pl.DeviceIdType.MESH)
            tx_wq.wait_send()
            tx_wo.wait_send()

    return pl.pallas_call(
        body,
        out_shape=jax.ShapeDtypeStruct((B_LOC, SQ, D_MODEL), jnp.float32),
        in_specs=[pl.BlockSpec(memory_space=pltpu.VMEM)] * 5,
        out_specs=pl.BlockSpec(memory_space=pltpu.VMEM),
        scratch_shapes=[
            pltpu.VMEM((N_DEV, D_MODEL, D_CHUNK), jnp.float32),
            pltpu.VMEM((N_DEV, D_CHUNK, D_MODEL), jnp.float32),
            pltpu.SemaphoreType.DMA((N_DEV,)),
            pltpu.SemaphoreType.DMA((N_DEV,)),
            pltpu.SemaphoreType.DMA((N_DEV,)),
            pltpu.SemaphoreType.DMA((N_DEV,)),
        ],
        compiler_params=pltpu.CompilerParams(collective_id=0),
    )(x, Wq, K_t, V_t, Wo)


# device time: 120672 ns/iter; 1.7657x vs baseline; 1.7657x over previous
import jax
import jax.numpy as jnp
from jax import lax
from jax.experimental import pallas as pl
from jax.experimental.pallas import tpu as pltpu

N_DEV = 16
B_LOC = 2
SQ = 128
SKV = 128
D_MODEL = 512
H_LOC = 4
DH = 64
D_CHUNK = H_LOC * DH

CW_HOPS = 8
CCW_HOPS = 7


def _ring_to_logical(rr):
    cc = rr // 4
    pp = lax.rem(rr, 4)
    zz = jnp.where(lax.rem(cc, 2) == 0, pp, 3 - pp)
    return 4 * zz + cc


def _logical_to_ring(ll):
    cc = lax.rem(ll, 4)
    zz = ll // 4
    return 4 * cc + jnp.where(lax.rem(cc, 2) == 0, zz, 3 - zz)


def kernel(x, Wq, K_ext, V_ext, Wo):
    pos = lax.axis_index("i")
    K_loc = lax.dynamic_slice_in_dim(K_ext, pos * B_LOC, B_LOC, axis=0)
    V_loc = lax.dynamic_slice_in_dim(V_ext, pos * B_LOC, B_LOC, axis=0)
    K_t = jnp.transpose(K_loc, (2, 0, 1, 3))
    V_t = jnp.transpose(V_loc, (2, 0, 1, 3))

    def body(x_ref, wq_ref, k_ref, v_ref, wo_ref, out_ref,
             comm_wq, comm_wo, recv_wq, recv_wo,
             scw_wq, scw_wo, sccw_wq, sccw_wo):
        my = lax.axis_index("i")
        r = _logical_to_ring(my)
        right = _ring_to_logical(lax.rem(r + 1, N_DEV))
        left = _ring_to_logical(lax.rem(r - 1 + N_DEV, N_DEV))

        barrier = pltpu.get_barrier_semaphore()
        for nbr in (left, right):
            pl.semaphore_signal(barrier, inc=1, device_id=(nbr,),
                                device_id_type=pl.DeviceIdType.MESH)
        pl.semaphore_wait(barrier, 2)

        qblk = lax.broadcasted_iota(jnp.int32, (SQ, SKV), 0) // 64
        kblk = lax.broadcasted_iota(jnp.int32, (SQ, SKV), 1) // 64
        mask = kblk <= qblk

        x2 = x_ref[...].reshape(B_LOC * SQ, D_MODEL)

        def compute_chunk(o, wq_c, wo_c):
            q = jnp.dot(x2, wq_c, preferred_element_type=jnp.float32)
            k_c = k_ref[pl.ds(o * H_LOC, H_LOC)]
            v_c = v_ref[pl.ds(o * H_LOC, H_LOC)]
            for b in range(B_LOC):
                qb = q[b * SQ:(b + 1) * SQ]
                parts = []
                for h in range(H_LOC):
                    qh = qb[:, h * DH:(h + 1) * DH]
                    kh = k_c[h, b]
                    vh = v_c[h, b]
                    s = lax.dot_general(
                        qh, kh, (((1,), (1,)), ((), ())),
                        preferred_element_type=jnp.float32) * 0.125
                    s = jnp.where(mask, s, -1e9)
                    m = jnp.max(s, axis=1, keepdims=True)
                    w = jnp.exp(s - m)
                    w = w / jnp.sum(w, axis=1, keepdims=True)
                    parts.append(lax.dot_general(
                        w, vh, (((1,), (0,)), ((), ())),
                        preferred_element_type=jnp.float32))
                ctx = jnp.concatenate(parts, axis=1)
                out_ref[b] += jnp.dot(ctx, wo_c,
                                      preferred_element_type=jnp.float32)

        out_ref[...] = jnp.zeros_like(out_ref)

        def send_own(dst, ssem_wq, ssem_wo):
            a = pltpu.make_async_remote_copy(
                src_ref=wq_ref, dst_ref=comm_wq.at[my],
                send_sem=ssem_wq, recv_sem=recv_wq.at[my],
                device_id=(dst,), device_id_type=pl.DeviceIdType.MESH)
            b = pltpu.make_async_remote_copy(
                src_ref=wo_ref, dst_ref=comm_wo.at[my],
                send_sem=ssem_wo, recv_sem=recv_wo.at[my],
                device_id=(dst,), device_id_type=pl.DeviceIdType.MESH)
            a.start()
            b.start()

        def hop_descs(o, dst, ssem_wq, ssem_wo):
            a = pltpu.make_async_remote_copy(
                src_ref=comm_wq.at[o], dst_ref=comm_wq.at[o],
                send_sem=ssem_wq, recv_sem=recv_wq.at[o],
                device_id=(dst,), device_id_type=pl.DeviceIdType.MESH)
            b = pltpu.make_async_remote_copy(
                src_ref=comm_wo.at[o], dst_ref=comm_wo.at[o],
                send_sem=ssem_wo, recv_sem=recv_wo.at[o],
                device_id=(dst,), device_id_type=pl.DeviceIdType.MESH)
            return a, b

        send_own(right, scw_wq.at[0], scw_wo.at[0])
        send_own(left, sccw_wq.at[0], sccw_wo.at[0])

        compute_chunk(my, wq_ref[...], wo_ref[...])

        for h in range(1, CW_HOPS + 1):
            o = _ring_to_logical(lax.rem(r - h + N_DEV, N_DEV))
            cw_wq, cw_wo = hop_descs(o, right, scw_wq.at[h], scw_wo.at[h])
            cw_wq.wait_recv()
            cw_wo.wait_recv()
            if h < CW_HOPS:
                cw_wq.start()
                cw_wo.start()
            if h <= CCW_HOPS:
                o2 = _ring_to_logical(lax.rem(r + h, N_DEV))
                ccw_wq, ccw_wo = hop_descs(
                    o2, left, sccw_wq.at[h], sccw_wo.at[h])
                ccw_wq.wait_recv()
                ccw_wo.wait_recv()
                if h < CCW_HOPS:
                    ccw_wq.start()
                    ccw_wo.start()
            compute_chunk(o, comm_wq[o], comm_wo[o])
            if h <= CCW_HOPS:
                compute_chunk(o2, comm_wq[o2], comm_wo[o2])

        for h in range(CW_HOPS):
            a, b = hop_descs(0, right, scw_wq.at[h], scw_wo.at[h])
            a.wait_send()
            b.wait_send()
        for h in range(CCW_HOPS):
            a, b = hop_descs(0, left, sccw_wq.at[h], sccw_wo.at[h])
            a.wait_send()
            b.wait_send()

    return pl.pallas_call(
        body,
        out_shape=jax.ShapeDtypeStruct((B_LOC, SQ, D_MODEL), jnp.float32),
        in_specs=[pl.BlockSpec(memory_space=pltpu.VMEM)] * 5,
        out_specs=pl.BlockSpec(memory_space=pltpu.VMEM),
        scratch_shapes=[
            pltpu.VMEM((N_DEV, D_MODEL, D_CHUNK), jnp.float32),
            pltpu.VMEM((N_DEV, D_CHUNK, D_MODEL), jnp.float32),
            pltpu.SemaphoreType.DMA((N_DEV,)),
            pltpu.SemaphoreType.DMA((N_DEV,)),
            pltpu.SemaphoreType.DMA((CW_HOPS,)),
            pltpu.SemaphoreType.DMA((CW_HOPS,)),
            pltpu.SemaphoreType.DMA((CCW_HOPS,)),
            pltpu.SemaphoreType.DMA((CCW_HOPS,)),
        ],
        compiler_params=pltpu.CompilerParams(collective_id=0),
    )(x, Wq, K_t, V_t, Wo)


# device time: 75690 ns/iter; 2.8151x vs baseline; 1.5943x over previous
import jax
import jax.numpy as jnp
from jax import lax
from jax.experimental import pallas as pl
from jax.experimental.pallas import tpu as pltpu

N_DEV = 16
B_LOC = 2
SQ = 128
SKV = 128
D_MODEL = 512
H_LOC = 4
DH = 64
D_CHUNK = H_LOC * DH

CW_HOPS = 8
CCW_HOPS = 7


def _ring_to_logical(rr):
    cc = rr // 4
    pp = lax.rem(rr, 4)
    zz = jnp.where(lax.rem(cc, 2) == 0, pp, 3 - pp)
    return 4 * zz + cc


def _logical_to_ring(ll):
    cc = lax.rem(ll, 4)
    zz = ll // 4
    return 4 * cc + jnp.where(lax.rem(cc, 2) == 0, zz, 3 - zz)


def kernel(x, Wq, K_ext, V_ext, Wo):
    pos = lax.axis_index("i")
    K_loc = lax.dynamic_slice_in_dim(K_ext, pos * B_LOC, B_LOC, axis=0)
    V_loc = lax.dynamic_slice_in_dim(V_ext, pos * B_LOC, B_LOC, axis=0)
    K_t = jnp.transpose(K_loc, (2, 0, 1, 3))
    V_t = jnp.transpose(V_loc, (2, 0, 1, 3))
    Wq = Wq.astype(jnp.bfloat16)
    Wo = Wo.astype(jnp.bfloat16)

    def body(x_ref, wq_ref, k_ref, v_ref, wo_ref, out_ref,
             comm_wq, comm_wo, recv_wq, recv_wo,
             scw_wq, scw_wo, sccw_wq, sccw_wo):
        my = lax.axis_index("i")
        r = _logical_to_ring(my)
        right = _ring_to_logical(lax.rem(r + 1, N_DEV))
        left = _ring_to_logical(lax.rem(r - 1 + N_DEV, N_DEV))

        barrier = pltpu.get_barrier_semaphore()
        for nbr in (left, right):
            pl.semaphore_signal(barrier, inc=1, device_id=(nbr,),
                                device_id_type=pl.DeviceIdType.MESH)
        pl.semaphore_wait(barrier, 2)

        qblk = lax.broadcasted_iota(jnp.int32, (SQ, SKV), 0) // 64
        kblk = lax.broadcasted_iota(jnp.int32, (SQ, SKV), 1) // 64
        mask = kblk <= qblk

        x2 = x_ref[...].reshape(B_LOC * SQ, D_MODEL).astype(jnp.bfloat16)

        def compute_chunk(o, wq_c, wo_c):
            q = jnp.dot(x2, wq_c, preferred_element_type=jnp.float32)
            k_c = k_ref[pl.ds(o * H_LOC, H_LOC)]
            v_c = v_ref[pl.ds(o * H_LOC, H_LOC)]
            for b in range(B_LOC):
                qb = q[b * SQ:(b + 1) * SQ]
                parts = []
                for h in range(H_LOC):
                    qh = qb[:, h * DH:(h + 1) * DH]
                    kh = k_c[h, b]
                    vh = v_c[h, b]
                    s = lax.dot_general(
                        qh, kh, (((1,), (1,)), ((), ())),
                        preferred_element_type=jnp.float32) * 0.125
                    s = jnp.where(mask, s, -1e9)
                    m = jnp.max(s, axis=1, keepdims=True)
                    w = jnp.exp(s - m)
                    w = w / jnp.sum(w, axis=1, keepdims=True)
                    parts.append(lax.dot_general(
                        w, vh, (((1,), (0,)), ((), ())),
                        preferred_element_type=jnp.float32))
                ctx = jnp.concatenate(parts, axis=1).astype(jnp.bfloat16)
                out_ref[b] += jnp.dot(ctx, wo_c,
                                      preferred_element_type=jnp.float32)

        out_ref[...] = jnp.zeros_like(out_ref)

        def send_own(dst, ssem_wq, ssem_wo):
            a = pltpu.make_async_remote_copy(
                src_ref=wq_ref, dst_ref=comm_wq.at[my],
                send_sem=ssem_wq, recv_sem=recv_wq.at[my],
                device_id=(dst,), device_id_type=pl.DeviceIdType.MESH)
            b = pltpu.make_async_remote_copy(
                src_ref=wo_ref, dst_ref=comm_wo.at[my],
                send_sem=ssem_wo, recv_sem=recv_wo.at[my],
                device_id=(dst,), device_id_type=pl.DeviceIdType.MESH)
            a.start()
            b.start()

        def hop_descs(o, dst, ssem_wq, ssem_wo):
            a = pltpu.make_async_remote_copy(
                src_ref=comm_wq.at[o], dst_ref=comm_wq.at[o],
                send_sem=ssem_wq, recv_sem=recv_wq.at[o],
                device_id=(dst,), device_id_type=pl.DeviceIdType.MESH)
            b = pltpu.make_async_remote_copy(
                src_ref=comm_wo.at[o], dst_ref=comm_wo.at[o],
                send_sem=ssem_wo, recv_sem=recv_wo.at[o],
                device_id=(dst,), device_id_type=pl.DeviceIdType.MESH)
            return a, b

        send_own(right, scw_wq.at[0], scw_wo.at[0])
        send_own(left, sccw_wq.at[0], sccw_wo.at[0])

        compute_chunk(my, wq_ref[...], wo_ref[...])

        for h in range(1, CW_HOPS + 1):
            o = _ring_to_logical(lax.rem(r - h + N_DEV, N_DEV))
            cw_wq, cw_wo = hop_descs(o, right, scw_wq.at[h], scw_wo.at[h])
            cw_wq.wait_recv()
            cw_wo.wait_recv()
            if h < CW_HOPS:
                cw_wq.start()
                cw_wo.start()
            if h <= CCW_HOPS:
                o2 = _ring_to_logical(lax.rem(r + h, N_DEV))
                ccw_wq, ccw_wo = hop_descs(
                    o2, left, sccw_wq.at[h], sccw_wo.at[h])
                ccw_wq.wait_recv()
                ccw_wo.wait_recv()
                if h < CCW_HOPS:
                    ccw_wq.start()
                    ccw_wo.start()
            compute_chunk(o, comm_wq[o], comm_wo[o])
            if h <= CCW_HOPS:
                compute_chunk(o2, comm_wq[o2], comm_wo[o2])

        for h in range(CW_HOPS):
            a, b = hop_descs(0, right, scw_wq.at[h], scw_wo.at[h])
            a.wait_send()
            b.wait_send()
        for h in range(CCW_HOPS):
            a, b = hop_descs(0, left, sccw_wq.at[h], sccw_wo.at[h])
            a.wait_send()
            b.wait_send()

    return pl.pallas_call(
        body,
        out_shape=jax.ShapeDtypeStruct((B_LOC, SQ, D_MODEL), jnp.float32),
        in_specs=[pl.BlockSpec(memory_space=pltpu.VMEM)] * 5,
        out_specs=pl.BlockSpec(memory_space=pltpu.VMEM),
        scratch_shapes=[
            pltpu.VMEM((N_DEV, D_MODEL, D_CHUNK), jnp.bfloat16),
            pltpu.VMEM((N_DEV, D_CHUNK, D_MODEL), jnp.bfloat16),
            pltpu.SemaphoreType.DMA((N_DEV,)),
            pltpu.SemaphoreType.DMA((N_DEV,)),
            pltpu.SemaphoreType.DMA((CW_HOPS,)),
            pltpu.SemaphoreType.DMA((CW_HOPS,)),
            pltpu.SemaphoreType.DMA((CCW_HOPS,)),
            pltpu.SemaphoreType.DMA((CCW_HOPS,)),
        ],
        compiler_params=pltpu.CompilerParams(collective_id=0),
    )(x, Wq, K_t, V_t, Wo)


# device time: 68444 ns/iter; 3.1131x vs baseline; 1.1059x over previous
import jax
import jax.numpy as jnp
from jax import lax
from jax.experimental import pallas as pl
from jax.experimental.pallas import tpu as pltpu

N_DEV = 16
B_LOC = 2
SQ = 128
SKV = 128
D_MODEL = 512
H_LOC = 4
DH = 64
D_CHUNK = H_LOC * DH

CW_HOPS = 8
CCW_HOPS = 7


def _ring_to_logical(rr):
    cc = rr // 4
    pp = lax.rem(rr, 4)
    zz = jnp.where(lax.rem(cc, 2) == 0, pp, 3 - pp)
    return 4 * zz + cc


def _logical_to_ring(ll):
    cc = lax.rem(ll, 4)
    zz = ll // 4
    return 4 * cc + jnp.where(lax.rem(cc, 2) == 0, zz, 3 - zz)


def kernel(x, Wq, K_ext, V_ext, Wo):
    pos = lax.axis_index("i")
    K_loc = lax.dynamic_slice_in_dim(K_ext, pos * B_LOC, B_LOC, axis=0)
    V_loc = lax.dynamic_slice_in_dim(V_ext, pos * B_LOC, B_LOC, axis=0)
    K_t = jnp.transpose(K_loc, (2, 0, 1, 3))
    V_t = jnp.transpose(V_loc, (2, 0, 1, 3))
    Wq = Wq.astype(jnp.bfloat16)
    Wo = Wo.astype(jnp.bfloat16)

    def body(x_ref, wq_ref, k_ref, v_ref, wo_ref, out_ref,
             comm_wq, comm_wo, recv_wq, recv_wo,
             scw_wq, scw_wo, sccw_wq, sccw_wo):
        my = lax.axis_index("i")
        r = _logical_to_ring(my)
        right = _ring_to_logical(lax.rem(r + 1, N_DEV))
        left = _ring_to_logical(lax.rem(r - 1 + N_DEV, N_DEV))

        barrier = pltpu.get_barrier_semaphore()
        for nbr in (left, right):
            pl.semaphore_signal(barrier, inc=1, device_id=(nbr,),
                                device_id_type=pl.DeviceIdType.MESH)
        pl.semaphore_wait(barrier, 2)

        qblk = lax.broadcasted_iota(jnp.int32, (SQ, SKV), 0) // 64
        kblk = lax.broadcasted_iota(jnp.int32, (SQ, SKV), 1) // 64
        mask = kblk <= qblk

        x2 = x_ref[...].reshape(B_LOC * SQ, D_MODEL).astype(jnp.bfloat16)

        def compute_chunk(o, wq_c, wo_c):
            q = jnp.dot(x2, wq_c, preferred_element_type=jnp.float32)
            k_c = k_ref[pl.ds(o * H_LOC, H_LOC)]
            v_c = v_ref[pl.ds(o * H_LOC, H_LOC)]
            for b in range(B_LOC):
                qb = q[b * SQ:(b + 1) * SQ]
                parts = []
                for h in range(H_LOC):
                    qh = qb[:, h * DH:(h + 1) * DH]
                    kh = k_c[h, b]
                    vh = v_c[h, b]
                    s = lax.dot_general(
                        qh, kh, (((1,), (1,)), ((), ())),
                        preferred_element_type=jnp.float32) * 0.125
                    s = jnp.where(mask, s, -1e9)
                    m = jnp.max(s, axis=1, keepdims=True)
                    w = jnp.exp(s - m)
                    w = w / jnp.sum(w, axis=1, keepdims=True)
                    parts.append(lax.dot_general(
                        w, vh, (((1,), (0,)), ((), ())),
                        preferred_element_type=jnp.float32))
                ctx = jnp.concatenate(parts, axis=1).astype(jnp.bfloat16)
                out_ref[b] += jnp.dot(ctx, wo_c,
                                      preferred_element_type=jnp.float32)

        out_ref[...] = jnp.zeros_like(out_ref)

        def send_own(dst, ssem_wq, ssem_wo):
            a = pltpu.make_async_remote_copy(
                src_ref=wq_ref, dst_ref=comm_wq.at[my],
                send_sem=ssem_wq, recv_sem=recv_wq.at[my],
                device_id=(dst,), device_id_type=pl.DeviceIdType.MESH)
            b = pltpu.make_async_remote_copy(
                src_ref=wo_ref, dst_ref=comm_wo.at[my],
                send_sem=ssem_wo, recv_sem=recv_wo.at[my],
                device_id=(dst,), device_id_type=pl.DeviceIdType.MESH)
            a.start()
            b.start()

        def hop_descs(o, dst, ssem_wq, ssem_wo):
            a = pltpu.make_async_remote_copy(
                src_ref=comm_wq.at[o], dst_ref=comm_wq.at[o],
                send_sem=ssem_wq, recv_sem=recv_wq.at[o],
                device_id=(dst,), device_id_type=pl.DeviceIdType.MESH)
            b = pltpu.make_async_remote_copy(
                src_ref=comm_wo.at[o], dst_ref=comm_wo.at[o],
                send_sem=ssem_wo, recv_sem=recv_wo.at[o],
                device_id=(dst,), device_id_type=pl.DeviceIdType.MESH)
            return a, b

        send_own(right, scw_wq.at[0], scw_wo.at[0])
        send_own(left, sccw_wq.at[0], sccw_wo.at[0])

        compute_chunk(my, wq_ref[...], wo_ref[...])

        for h in range(1, CW_HOPS + 1):
            o = _ring_to_logical(lax.rem(r - h + N_DEV, N_DEV))
            cw_wq, cw_wo = hop_descs(o, right, scw_wq.at[h], scw_wo.at[h])
            if h <= CCW_HOPS:
                o2 = _ring_to_logical(lax.rem(r + h, N_DEV))
                ccw_wq, ccw_wo = hop_descs(
                    o2, left, sccw_wq.at[h], sccw_wo.at[h])
            cw_wq.wait_recv()
            if h < CW_HOPS:
                cw_wq.start()
            if h <= CCW_HOPS:
                ccw_wq.wait_recv()
                if h < CCW_HOPS:
                    ccw_wq.start()
            cw_wo.wait_recv()
            if h < CW_HOPS:
                cw_wo.start()
            if h <= CCW_HOPS:
                ccw_wo.wait_recv()
                if h < CCW_HOPS:
                    ccw_wo.start()
            compute_chunk(o, comm_wq[o], comm_wo[o])
            if h <= CCW_HOPS:
                compute_chunk(o2, comm_wq[o2], comm_wo[o2])

        for h in range(CW_HOPS):
            a, b = hop_descs(0, right, scw_wq.at[h], scw_wo.at[h])
            a.wait_send()
            b.wait_send()
        for h in range(CCW_HOPS):
            a, b = hop_descs(0, left, sccw_wq.at[h], sccw_wo.at[h])
            a.wait_send()
            b.wait_send()

    return pl.pallas_call(
        body,
        out_shape=jax.ShapeDtypeStruct((B_LOC, SQ, D_MODEL), jnp.float32),
        in_specs=[pl.BlockSpec(memory_space=pltpu.VMEM)] * 5,
        out_specs=pl.BlockSpec(memory_space=pltpu.VMEM),
        scratch_shapes=[
            pltpu.VMEM((N_DEV, D_MODEL, D_CHUNK), jnp.bfloat16),
            pltpu.VMEM((N_DEV, D_CHUNK, D_MODEL), jnp.bfloat16),
            pltpu.SemaphoreType.DMA((N_DEV,)),
            pltpu.SemaphoreType.DMA((N_DEV,)),
            pltpu.SemaphoreType.DMA((CW_HOPS,)),
            pltpu.SemaphoreType.DMA((CW_HOPS,)),
            pltpu.SemaphoreType.DMA((CCW_HOPS,)),
            pltpu.SemaphoreType.DMA((CCW_HOPS,)),
        ],
        compiler_params=pltpu.CompilerParams(collective_id=0),
    )(x, Wq, K_t, V_t, Wo)
